# baseline (speedup 1.0000x reference)
import functools

import jax
import jax.numpy as jnp
from jax import lax
from jax.experimental import pallas as pl
from jax.experimental.pallas import tpu as pltpu
from jax.experimental.pallas import tpu_sc as plsc

BATCH = 16384
HIST = 50
D = 16
N_FLAT = BATCH * HIST

NC = 2
NS = 16
NW = NC * NS
CPW = BATCH // 128 // NW
TOK = CPW * 128

_mesh = plsc.VectorSubcoreMesh(core_axis_name="c", subcore_axis_name="s")


@functools.partial(
    pl.kernel,
    mesh=_mesh,
    out_type=jax.ShapeDtypeStruct((HIST * 128, 128 * D), jnp.float32),
    scratch_types=[
        pltpu.VMEM((HIST * TOK,), jnp.int32),
        pltpu.VMEM((TOK, D), jnp.float32),
        pltpu.VMEM((TOK, D), jnp.float32),
        pltpu.VMEM((TOK, D), jnp.float32),
        pltpu.VMEM((TOK, D), jnp.float32),
        pltpu.SemaphoreType.DMA,
        pltpu.SemaphoreType.DMA,
        pltpu.SemaphoreType.DMA,
        pltpu.SemaphoreType.DMA,
        pltpu.SemaphoreType.DMA,
        pltpu.SemaphoreType.DMA,
        pltpu.SemaphoreType.DMA,
        pltpu.SemaphoreType.DMA,
        pltpu.SemaphoreType.DMA,
    ],
    compiler_params=pltpu.CompilerParams(use_tc_tiling_on_sc=False),
)
def _gather_kernel(idx_hbm, table_hbm, out_hbm, idx_v, rows0, rows1,
                   rows2, rows3, isem, gs0, gs1, gs2, gs3,
                   os0, os1, os2, os3):
    wid = lax.axis_index("s") * NC + lax.axis_index("c")
    c0 = wid * CPW
    rows = (rows0, rows1, rows2, rows3)
    gsem = (gs0, gs1, gs2, gs3)
    osem = (os0, os1, os2, os3)

    idx_copies = []
    for h in range(HIST):
        off = pl.multiple_of(h * BATCH + c0 * 128, TOK)
        idx_copies.append(pltpu.async_copy(
            idx_hbm.at[pl.ds(off, TOK)],
            idx_v.at[pl.ds(h * TOK, TOK)], isem))
    for c in idx_copies:
        c.wait()

    NB = 4

    def start_gather(h):
        return pltpu.async_copy(
            table_hbm.at[idx_v.at[pl.ds(h * TOK, TOK)]],
            rows[h % NB], gsem[h % NB])

    def start_outs(h):
        p = h % NB
        base = pl.multiple_of(h * 128, 128)
        cps = []
        for cp in range(CPW):
            cps.append(pltpu.async_copy(
                rows[p].at[pl.ds(cp * 128, 128), :],
                out_hbm.at[pl.ds(base, 128), pl.ds((c0 + cp) * D, D)],
                osem[p]))
        return cps

    gathers = [None] * HIST
    outs = [None] * HIST
    for h in range(NB - 1):
        gathers[h] = start_gather(h)
    for h in range(HIST):
        if h + NB - 1 < HIST:
            if h >= 1:
                for c in outs[h - 1]:
                    c.wait()
            gathers[h + NB - 1] = start_gather(h + NB - 1)
        gathers[h].wait()
        outs[h] = start_outs(h)
    for hh in range(HIST - NB, HIST):
        for c in outs[hh]:
            c.wait()


def _tc_transpose_body(i_ref, o_ref):
    x = i_ref[0]
    for g in range(2):
        o_ref[0, g] = x[:, :, g, :].transpose(1, 2, 0).reshape(128, 128)


_tc_transpose = pl.pallas_call(
    _tc_transpose_body,
    grid=(HIST, 8),
    in_specs=[pl.BlockSpec((1, 128, 16, 2, 8), lambda h, j: (h, 0, j, 0, 0))],
    out_specs=pl.BlockSpec((1, 2, 128, 128), lambda h, j: (h, 0, j, 0)),
    out_shape=jax.ShapeDtypeStruct((HIST, 2, 1024, 128), jnp.float32),
)


def kernel(x, table):
    idx = x.T.reshape(N_FLAT).astype(jnp.int32)
    slabs = _gather_kernel(idx, table)
    out2 = _tc_transpose(slabs.reshape(HIST, 128, 128, 2, 8))
    return (out2.reshape(HIST, 2, 128, 8, 128)
            .transpose(2, 4, 0, 1, 3).reshape(BATCH, HIST, D))


# speedup vs baseline: 4.1131x; 4.1131x over previous
import functools

import jax
import jax.numpy as jnp
from jax import lax
from jax.experimental import pallas as pl
from jax.experimental.pallas import tpu as pltpu
from jax.experimental.pallas import tpu_sc as plsc

BATCH = 16384
HIST = 50
D = 16
N_FLAT = BATCH * HIST

NC = 2
NS = 16
NW = NC * NS
CPW = BATCH // 128 // NW
TOK = CPW * 128

_mesh = plsc.VectorSubcoreMesh(core_axis_name="c", subcore_axis_name="s")


@functools.partial(
    pl.kernel,
    mesh=_mesh,
    out_type=jax.ShapeDtypeStruct((HIST * 2 * 128, 1024), jnp.float32),
    scratch_types=[
        pltpu.VMEM((HIST * TOK,), jnp.int32),
        pltpu.VMEM((TOK, D), jnp.float32),
        pltpu.VMEM((TOK, D), jnp.float32),
        pltpu.VMEM((TOK, D), jnp.float32),
        pltpu.VMEM((TOK, D), jnp.float32),
        pltpu.SemaphoreType.DMA,
        pltpu.SemaphoreType.DMA,
        pltpu.SemaphoreType.DMA,
        pltpu.SemaphoreType.DMA,
        pltpu.SemaphoreType.DMA,
        pltpu.SemaphoreType.DMA,
        pltpu.SemaphoreType.DMA,
        pltpu.SemaphoreType.DMA,
        pltpu.SemaphoreType.DMA,
    ],
    compiler_params=pltpu.CompilerParams(use_tc_tiling_on_sc=False),
)
def _gather_kernel(idx_hbm, table_hbm, out_hbm, idx_v, rows0, rows1,
                   rows2, rows3, isem, gs0, gs1, gs2, gs3,
                   os0, os1, os2, os3):
    wid = lax.axis_index("s") * NC + lax.axis_index("c")
    c0 = wid * CPW
    rows = (rows0, rows1, rows2, rows3)
    gsem = (gs0, gs1, gs2, gs3)
    osem = (os0, os1, os2, os3)

    idx_copies = []
    for h in range(HIST):
        off = pl.multiple_of(h * BATCH + c0 * 128, TOK)
        idx_copies.append(pltpu.async_copy(
            idx_hbm.at[pl.ds(off, TOK)],
            idx_v.at[pl.ds(h * TOK, TOK)], isem))
    for c in idx_copies:
        c.wait()

    NB = 4

    def start_gather(h):
        return pltpu.async_copy(
            table_hbm.at[idx_v.at[pl.ds(h * TOK, TOK)]],
            rows[h % NB], gsem[h % NB])

    def start_outs(h):
        p = h % NB
        cps = []
        for g in range(2):
            base = pl.multiple_of(h * 256 + g * 128, 128)
            for cp in range(CPW):
                cps.append(pltpu.async_copy(
                    rows[p].at[pl.ds(cp * 128, 128), pl.ds(g * 8, 8)],
                    out_hbm.at[pl.ds(base, 128), pl.ds((c0 + cp) * 8, 8)],
                    osem[p]))
        return cps

    gathers = [None] * HIST
    outs = [None] * HIST
    for h in range(NB - 1):
        gathers[h] = start_gather(h)
    for h in range(HIST):
        if h + NB - 1 < HIST:
            if h >= 1:
                for c in outs[h - 1]:
                    c.wait()
            gathers[h + NB - 1] = start_gather(h + NB - 1)
        gathers[h].wait()
        outs[h] = start_outs(h)
    for hh in range(HIST - NB, HIST):
        for c in outs[hh]:
            c.wait()


def _tc_transpose_body(i_ref, o_ref):
    o_ref[...] = i_ref[...].T


_tc_transpose = pl.pallas_call(
    _tc_transpose_body,
    grid=(HIST * 2,),
    in_specs=[pl.BlockSpec((128, 1024), lambda i: (i, 0))],
    out_specs=pl.BlockSpec((1024, 128), lambda i: (i, 0)),
    out_shape=jax.ShapeDtypeStruct((HIST * 2 * 1024, 128), jnp.float32),
)


def kernel(x, table):
    idx = x.T.reshape(N_FLAT).astype(jnp.int32)
    slabs = _gather_kernel(idx, table)
    out2 = _tc_transpose(slabs)
    return (out2.reshape(HIST, 2, 128, 8, 128)
            .transpose(2, 4, 0, 1, 3).reshape(BATCH, HIST, D))
